# baseline (speedup 1.0000x reference)
import math

import jax
import jax.numpy as jnp
from jax.experimental import pallas as pl
from jax.experimental.pallas import tpu as pltpu

_NUM_CLASS = 2
_STEP = 12304
_TOTAL_STEP = 20000
_MARGIN = 3.0
_DEC = math.pow(10.0, -_STEP / _TOTAL_STEP)
_MINP = 0.0001


def _aaf_kernel(we_ref, wne_ref, pred_ref, lab_ref, out_ref, acc_ref):
    n = pl.program_id(0)
    num_n = pl.num_programs(0)
    nb = pred_ref.shape[0]
    ih = pred_ref.shape[1]
    h = nb * ih
    w = pred_ref.shape[2]
    hm = _MARGIN / 2.0
    l0 = math.log(_MINP)
    bf = jnp.bfloat16

    @pl.when(n == 0)
    def _init():
        for si in range(3):
            for k in range(3):
                acc_ref[si, k] = jnp.float32(0.0)

    lab = lab_ref[...].reshape(h, w)
    p = jnp.clip(pred_ref[...].reshape(h, w), _MINP, 1.0)
    lp = jnp.log(p)

    lab_b = lab.astype(bf)
    lab16 = lab.astype(jnp.int16)
    lp_b = lp.astype(bf)
    p_b = p.astype(bf)

    kl_pad = (2.0 * _MINP) * (l0 - lp)

    one_b = jnp.ones((), bf)
    zero_b = jnp.zeros((), bf)
    hm_b = jnp.full((), hm, bf)
    nhm_b = jnp.full((), -hm, bf)

    rows16 = jax.lax.broadcasted_iota(jnp.int16, (h, w), 0)
    cols16 = jax.lax.broadcasted_iota(jnp.int16, (h, w), 1)
    rows = jax.lax.broadcasted_iota(jnp.int32, (h, w), 0)
    cols = jax.lax.broadcasted_iota(jnp.int32, (h, w), 1)
    lrow16 = jnp.bitwise_and(rows16, jnp.int16(ih - 1))
    lrow = jnp.bitwise_and(rows, ih - 1)
    n0f = jnp.where(n == 0, 1.0, 0.0)
    is00 = jnp.logical_and(rows == 0, cols == 0).astype(jnp.float32)
    drop = is00 * n0f

    for si, s in enumerate((1, 2, 3)):
        lab_e = pltpu.roll(lab16, w - s, 1)
        lp_e = pltpu.roll(lp_b, w - s, 1)
        p_e = pltpu.roll(p_b, w - s, 1)
        lab_s = pltpu.roll(lab16, h - s, 0)
        lp_s = pltpu.roll(lp_b, h - s, 0)
        p_s = pltpu.roll(p_b, h - s, 0)
        lab_se = pltpu.roll(lab_s, w - s, 1)
        lp_se = pltpu.roll(lp_s, w - s, 1)
        p_se = pltpu.roll(p_s, w - s, 1)
        lab_sw = pltpu.roll(lab_s, s, 1)
        lp_sw = pltpu.roll(lp_s, s, 1)
        p_sw = pltpu.roll(p_s, s, 1)

        vrow = lrow16 < (ih - s)
        vcol_e = cols16 < (w - s)
        vcol_w = cols16 >= s
        dirs = (
            (lab_e, lp_e, p_e, vcol_e),
            (lab_s, lp_s, p_s, vrow),
            (lab_se, lp_se, p_se, jnp.logical_and(vrow, vcol_e)),
            (lab_sw, lp_sw, p_sw, jnp.logical_and(vrow, vcol_w)),
        )
        acc_cnt = jnp.zeros((h, w), bf)
        acc_f = jnp.zeros((h, w), bf)
        acc_rn = jnp.zeros((h, w), bf)
        acc_n = jnp.zeros((h, w), bf)
        for labg, lpg, pg, vmask in dirs:
            d = lpg - lp_b
            klf = pg * d
            tpdr = p_b * d
            er = labg != lab16
            e = jnp.logical_and(er, vmask)
            ne = jnp.logical_xor(vmask, e)
            acc_cnt += jnp.where(e, one_b, zero_b)
            acc_f += jnp.where(e, jnp.minimum(klf, hm_b), zero_b)
            acc_rn += jnp.where(e, jnp.maximum(tpdr, nhm_b), zero_b)
            acc_n += jnp.where(ne, klf - tpdr, zero_b)

        a = we_ref[0, si]
        b = we_ref[1, si] - we_ref[0, si]
        c = wne_ref[0, si]
        dd = wne_ref[1, si] - wne_ref[0, si]
        a_b = a.astype(bf)
        b_b = b.astype(bf)
        ab_b = (a + b).astype(bf)
        c_b = c.astype(bf)
        dd_b = dd.astype(bf)
        we_b = a_b + b_b * lab_b
        we_r_b = ab_b - b_b * lab_b
        wne_b = c_b + dd_b * lab_b

        rin = (3.0 - jnp.where(lrow < s, 1.0, 0.0)
               - jnp.where(lrow >= ih - s, 1.0, 0.0))
        cin = (3.0 - jnp.where(cols < s, 1.0, 0.0)
               - jnp.where(cols >= w - s, 1.0, 0.0))
        padcnt = 9.0 - rin * cin - drop
        pe = padcnt * lab
        s_pe = jnp.sum(pe)
        s_pekl = jnp.sum(pe * kl_pad)
        s_pckl = jnp.sum(padcnt * kl_pad)
        cnt_int = jnp.sum(acc_cnt, dtype=jnp.float32)

        m = _MARGIN
        sum_e = (m * (2.0 * a + b) * cnt_int
                 - 2.0 * jnp.sum(we_b * acc_f, dtype=jnp.float32)
                 + 2.0 * jnp.sum(we_r_b * acc_rn, dtype=jnp.float32)
                 + (a + b) * (m * s_pe - s_pekl))
        sum_ne = (2.0 * jnp.sum(wne_b * acc_n, dtype=jnp.float32)
                  + c * (s_pckl - s_pekl))
        cnt_e = 2.0 * cnt_int + s_pe

        acc_ref[si, 0] += sum_e
        acc_ref[si, 1] += cnt_e
        acc_ref[si, 2] += sum_ne

    @pl.when(n == num_n - 1)
    def _fin():
        total = 8.0 * num_n * h * w
        aaf = jnp.float32(0.0)
        for si in range(3):
            se = acc_ref[si, 0]
            ce = acc_ref[si, 1]
            sne = acc_ref[si, 2]
            cne = total - 1.0 - ce
            aaf = aaf + se / ce + sne / cne
        out_ref[0, 0] = aaf * _DEC


@jax.jit
def kernel(pred, gt, w_edge, w_not_edge):
    n, h, w, _ = pred.shape
    lab = gt[..., 0].astype(jnp.float32)
    pr = pred[..., 0]
    sw_e = jax.nn.softmax(w_edge.reshape(_NUM_CLASS, 3), axis=-1)
    sw_ne = jax.nn.softmax(w_not_edge.reshape(_NUM_CLASS, 3), axis=-1)
    nb = 2 if n % 2 == 0 else 1
    out = pl.pallas_call(
        _aaf_kernel,
        grid=(n // nb,),
        in_specs=[
            pl.BlockSpec(memory_space=pltpu.SMEM),
            pl.BlockSpec(memory_space=pltpu.SMEM),
            pl.BlockSpec((nb, h, w), lambda i: (i, 0, 0)),
            pl.BlockSpec((nb, h, w), lambda i: (i, 0, 0)),
        ],
        out_specs=pl.BlockSpec(memory_space=pltpu.SMEM),
        out_shape=jax.ShapeDtypeStruct((1, 1), jnp.float32),
        scratch_shapes=[pltpu.SMEM((3, 4), jnp.float32)],
    )(sw_e, sw_ne, pr, lab)
    return out[0, 0]


# speedup vs baseline: 1.0084x; 1.0084x over previous
import math

import jax
import jax.numpy as jnp
from jax.experimental import pallas as pl
from jax.experimental.pallas import tpu as pltpu

_NUM_CLASS = 2
_STEP = 12304
_TOTAL_STEP = 20000
_MARGIN = 3.0
_DEC = math.pow(10.0, -_STEP / _TOTAL_STEP)
_MINP = 0.0001


def _aaf_kernel(we_ref, wne_ref, pred_ref, lab_ref, out_ref, acc_ref):
    n = pl.program_id(0)
    num_n = pl.num_programs(0)
    h = pred_ref.shape[1]
    w = pred_ref.shape[2]
    hm = _MARGIN / 2.0
    l0 = math.log(_MINP)
    bf = jnp.bfloat16

    @pl.when(n == 0)
    def _init():
        for si in range(3):
            for k in range(3):
                acc_ref[si, k] = jnp.float32(0.0)

    lab = lab_ref[0]
    p = jnp.clip(pred_ref[0], _MINP, 1.0)
    lp = jnp.log(p)

    lab_b = lab.astype(bf)
    lp_b = lp.astype(bf)
    p_b = p.astype(bf)

    kl_pad = (2.0 * _MINP) * (l0 - lp)

    one_b = jnp.ones((), bf)
    zero_b = jnp.zeros((), bf)
    hm_b = jnp.full((), hm, bf)
    nhm_b = jnp.full((), -hm, bf)

    rows16 = jax.lax.broadcasted_iota(jnp.int16, (h, w), 0)
    cols16 = jax.lax.broadcasted_iota(jnp.int16, (h, w), 1)
    rows = jax.lax.broadcasted_iota(jnp.int32, (h, w), 0)
    cols = jax.lax.broadcasted_iota(jnp.int32, (h, w), 1)
    n0f = jnp.where(n == 0, 1.0, 0.0)
    is00 = jnp.logical_and(rows == 0, cols == 0).astype(jnp.float32)
    drop = is00 * n0f

    for si, s in enumerate((1, 2, 3)):
        lab_e = pltpu.roll(lab_b, w - s, 1)
        lp_e = pltpu.roll(lp_b, w - s, 1)
        p_e = pltpu.roll(p_b, w - s, 1)
        lab_s = pltpu.roll(lab_b, h - s, 0)
        lp_s = pltpu.roll(lp_b, h - s, 0)
        p_s = pltpu.roll(p_b, h - s, 0)
        lab_se = pltpu.roll(lab_s, w - s, 1)
        lp_se = pltpu.roll(lp_s, w - s, 1)
        p_se = pltpu.roll(p_s, w - s, 1)
        lab_sw = pltpu.roll(lab_s, s, 1)
        lp_sw = pltpu.roll(lp_s, s, 1)
        p_sw = pltpu.roll(p_s, s, 1)

        vrow = rows16 < (h - s)
        vcol_e = cols16 < (w - s)
        vcol_w = cols16 >= s
        dirs = (
            (lab_e, lp_e, p_e, vcol_e),
            (lab_s, lp_s, p_s, vrow),
            (lab_se, lp_se, p_se, jnp.logical_and(vrow, vcol_e)),
            (lab_sw, lp_sw, p_sw, jnp.logical_and(vrow, vcol_w)),
        )
        acc_cnt = jnp.zeros((h, w), bf)
        acc_f = jnp.zeros((h, w), bf)
        acc_rn = jnp.zeros((h, w), bf)
        acc_n = jnp.zeros((h, w), bf)
        for labg, lpg, pg, vmask in dirs:
            d = lpg - lp_b
            klf = pg * d
            tpdr = p_b * d
            er = labg != lab_b
            e = jnp.logical_and(er, vmask)
            ne = jnp.logical_xor(vmask, e)
            acc_cnt += jnp.where(e, one_b, zero_b)
            acc_f += jnp.where(e, jnp.minimum(klf, hm_b), zero_b)
            acc_rn += jnp.where(e, jnp.maximum(tpdr, nhm_b), zero_b)
            acc_n += jnp.where(ne, klf - tpdr, zero_b)

        a = we_ref[0, si]
        b = we_ref[1, si] - we_ref[0, si]
        c = wne_ref[0, si]
        dd = wne_ref[1, si] - wne_ref[0, si]
        a_b = a.astype(bf)
        b_b = b.astype(bf)
        ab_b = (a + b).astype(bf)
        c_b = c.astype(bf)
        dd_b = dd.astype(bf)
        we_b = a_b + b_b * lab_b
        we_r_b = ab_b - b_b * lab_b
        wne_b = c_b + dd_b * lab_b

        rin = (3.0 - jnp.where(rows < s, 1.0, 0.0)
               - jnp.where(rows >= h - s, 1.0, 0.0))
        cin = (3.0 - jnp.where(cols < s, 1.0, 0.0)
               - jnp.where(cols >= w - s, 1.0, 0.0))
        padcnt = 9.0 - rin * cin - drop
        pe = padcnt * lab
        s_pe = jnp.sum(pe)
        s_pekl = jnp.sum(pe * kl_pad)
        s_pckl = jnp.sum(padcnt * kl_pad)
        cnt_int = jnp.sum(acc_cnt, dtype=jnp.float32)

        m = _MARGIN
        sum_e = (m * (2.0 * a + b) * cnt_int
                 - 2.0 * jnp.sum(we_b * acc_f, dtype=jnp.float32)
                 + 2.0 * jnp.sum(we_r_b * acc_rn, dtype=jnp.float32)
                 + (a + b) * (m * s_pe - s_pekl))
        sum_ne = (2.0 * jnp.sum(wne_b * acc_n, dtype=jnp.float32)
                  + c * (s_pckl - s_pekl))
        cnt_e = 2.0 * cnt_int + s_pe

        acc_ref[si, 0] += sum_e
        acc_ref[si, 1] += cnt_e
        acc_ref[si, 2] += sum_ne

    @pl.when(n == num_n - 1)
    def _fin():
        total = 8.0 * num_n * h * w
        aaf = jnp.float32(0.0)
        for si in range(3):
            se = acc_ref[si, 0]
            ce = acc_ref[si, 1]
            sne = acc_ref[si, 2]
            cne = total - 1.0 - ce
            aaf = aaf + se / ce + sne / cne
        out_ref[0, 0] = aaf * _DEC


@jax.jit
def kernel(pred, gt, w_edge, w_not_edge):
    n, h, w, _ = pred.shape
    lab = gt[..., 0].astype(jnp.float32)
    pr = pred[..., 0]
    sw_e = jax.nn.softmax(w_edge.reshape(_NUM_CLASS, 3), axis=-1)
    sw_ne = jax.nn.softmax(w_not_edge.reshape(_NUM_CLASS, 3), axis=-1)
    out = pl.pallas_call(
        _aaf_kernel,
        grid=(n,),
        in_specs=[
            pl.BlockSpec(memory_space=pltpu.SMEM),
            pl.BlockSpec(memory_space=pltpu.SMEM),
            pl.BlockSpec((1, h, w), lambda i: (i, 0, 0)),
            pl.BlockSpec((1, h, w), lambda i: (i, 0, 0)),
        ],
        out_specs=pl.BlockSpec(memory_space=pltpu.SMEM),
        out_shape=jax.ShapeDtypeStruct((1, 1), jnp.float32),
        scratch_shapes=[pltpu.SMEM((3, 4), jnp.float32)],
    )(sw_e, sw_ne, pr, lab)
    return out[0, 0]
